# baseline (speedup 1.0000x reference)
import functools

import jax
import jax.numpy as jnp
from jax import lax
from jax.experimental import pallas as pl
from jax.experimental.pallas import tpu as pltpu
from jax.experimental.pallas import tpu_sc as plsc

EMBEDDING_DIM = 64
BATCH_SIZE = 16384

NUM_CORES = 2
NUM_SUBCORES = 16
NUM_WORKERS = NUM_CORES * NUM_SUBCORES
B_PER_W = BATCH_SIZE // NUM_WORKERS
CHUNK = 128
NUM_CHUNKS = B_PER_W // CHUNK


@jax.jit
def _sc_gather(idx, table):
    mesh = plsc.VectorSubcoreMesh(core_axis_name="c", subcore_axis_name="s")

    @functools.partial(
        pl.kernel,
        out_type=jax.ShapeDtypeStruct((BATCH_SIZE, EMBEDDING_DIM), jnp.float32),
        mesh=mesh,
        scratch_types=[
            pltpu.VMEM((B_PER_W,), jnp.int32),
            pltpu.VMEM((B_PER_W, EMBEDDING_DIM), jnp.float32),
            pltpu.SemaphoreType.DMA,
        ],
        compiler_params=pltpu.CompilerParams(use_tc_tiling_on_sc=False),
    )
    def k(idx_hbm, table_hbm, out_hbm, idx_v, rows_v, sem):
        wid = lax.axis_index("s") * NUM_CORES + lax.axis_index("c")
        base = wid * B_PER_W
        pltpu.sync_copy(idx_hbm.at[pl.ds(base, B_PER_W)], idx_v)
        copies = []
        for j in range(NUM_CHUNKS):
            copies.append(
                pltpu.async_copy(
                    table_hbm.at[idx_v.at[pl.ds(j * CHUNK, CHUNK)]],
                    rows_v.at[pl.ds(j * CHUNK, CHUNK)],
                    sem,
                )
            )
        for c in copies:
            c.wait()
        pltpu.sync_copy(rows_v, out_hbm.at[pl.ds(base, B_PER_W)])

    return k(idx, table)


def kernel(inputs, embeddings):
    return _sc_gather(inputs.astype(jnp.int32), embeddings)


# speedup vs baseline: 1.1568x; 1.1568x over previous
import functools

import jax
import jax.numpy as jnp
from jax import lax
from jax.experimental import pallas as pl
from jax.experimental.pallas import tpu as pltpu
from jax.experimental.pallas import tpu_sc as plsc

NUM_EMBEDDINGS = 1000000
EMBEDDING_DIM = 64
BATCH_SIZE = 16384

NUM_CORES = 2
NUM_SUBCORES = 16
NUM_WORKERS = NUM_CORES * NUM_SUBCORES
B_PER_W = BATCH_SIZE // NUM_WORKERS
LANES = 16


@jax.jit
def _sc_gather(idx, table3):
    mesh = plsc.VectorSubcoreMesh(core_axis_name="c", subcore_axis_name="s")

    @functools.partial(
        pl.kernel,
        out_type=jax.ShapeDtypeStruct((BATCH_SIZE // 8, 8, EMBEDDING_DIM),
                                      jnp.float32),
        mesh=mesh,
        scratch_types=[
            pltpu.VMEM((B_PER_W,), jnp.int32),
            pltpu.VMEM((8, EMBEDDING_DIM), jnp.float32),
            pltpu.VMEM((4, 8, EMBEDDING_DIM), jnp.float32),
            pltpu.SemaphoreType.DMA,
        ],
        compiler_params=pltpu.CompilerParams(needs_layout_passes=False),
    )
    def k(idx_hbm, table_hbm, out_hbm, idx_v, tile_v, rows_v, sem):
        wid = lax.axis_index("s") * NUM_CORES + lax.axis_index("c")
        base = wid * B_PER_W
        pltpu.sync_copy(idx_hbm.at[pl.ds(base, B_PER_W)], idx_v)
        iota = lax.iota(jnp.int32, LANES)

        def group_body(g, _):
            v16 = idx_v[pl.ds(g * LANES, LANES)]
            t16 = v16 >> 3
            s16 = v16 & 7
            for l in range(LANES):
                tid = jnp.sum(jnp.where(iota == l, t16, 0))
                s = jnp.sum(jnp.where(iota == l, s16, 0))
                pltpu.async_copy(table_hbm.at[tid], tile_v, sem).wait()
                b = g * LANES + l
                for h in range(EMBEDDING_DIM // LANES):
                    rows_v[(b % 32) // 8, b % 8, pl.ds(h * LANES, LANES)] = (
                        tile_v[s, pl.ds(h * LANES, LANES)]
                    )
            return 0

        def chunk_body(ch, _):
            lax.fori_loop(ch * 2, ch * 2 + 2, group_body, 0)
            pltpu.sync_copy(
                rows_v, out_hbm.at[pl.ds(wid * 64 + ch * 4, 4)]
            )
            return 0

        lax.fori_loop(0, B_PER_W // 32, chunk_body, 0)

    return k(idx, table3)


def kernel(inputs, embeddings):
    table3 = embeddings.reshape(NUM_EMBEDDINGS // 8, 8, EMBEDDING_DIM)
    out3 = _sc_gather(inputs.astype(jnp.int32), table3)
    return out3.reshape(BATCH_SIZE, EMBEDDING_DIM)
